# baseline (speedup 1.0000x reference)
import functools

import jax
import jax.numpy as jnp
from jax import lax
from jax.experimental import pallas as pl
from jax.experimental.pallas import tpu as pltpu
from jax.experimental.pallas import tpu_sc as plsc

NUM_EMBEDDINGS = 1000000
D = 64
BATCH = 4096
SEQ = 200
B_TOTAL = BATCH * SEQ

NC, NS = 2, 16
NW = NC * NS
C = 128
Q_PER_W = (SEQ * 4) // NW
NCH = Q_PER_W * 8
S = 4
A = 2
L = 16


def _chunk_pos(w, t):
    qg = w * Q_PER_W + t // 8
    s = qg // 4
    boff = (qg % 4) * 1024 + (t % 8) * C
    return s, boff


@functools.partial(
    pl.kernel,
    out_type=jax.ShapeDtypeStruct((SEQ, D, BATCH), jnp.float32),
    mesh=plsc.VectorSubcoreMesh(core_axis_name="c", subcore_axis_name="s"),
    compiler_params=pltpu.CompilerParams(
        use_tc_tiling_on_sc=True, needs_layout_passes=False
    ),
    scratch_types=[
        [pltpu.VMEM((C,), jnp.int32) for _ in range(S)],
        [pltpu.VMEM((C,), jnp.int32) for _ in range(S)],
        [pltpu.VMEM((C, 2 * D), jnp.float32) for _ in range(S)],
        [pltpu.VMEM((D, C), jnp.float32) for _ in range(S)],
        [pltpu.SemaphoreType.DMA for _ in range(S)],
        [pltpu.SemaphoreType.DMA for _ in range(S)],
        [pltpu.SemaphoreType.DMA for _ in range(S)],
    ],
)
def _embed_sc(ids_hbm, table_hbm, out_hbm, rows, cols, gbuf, tbuf,
              isems, gsems, wsems):
    wid = lax.axis_index("s") * NC + lax.axis_index("c")

    def _flat_off(t):
        s, boff = _chunk_pos(wid, t)
        return s * BATCH + boff

    def _idx_start(t, b):
        pltpu.make_async_copy(
            ids_hbm.at[pl.ds(_flat_off(t), C)], rows[b], isems[b]
        ).start()

    def _idx_wait(t, b):
        pltpu.make_async_copy(
            ids_hbm.at[pl.ds(_flat_off(t), C)], rows[b], isems[b]
        ).wait()

    def _gather_start(t, b):
        for k in range(C // L):
            ids_v = rows[b][pl.ds(k * L, L)]
            rows[b][pl.ds(k * L, L)] = lax.shift_right_logical(ids_v, 1)
            cols[b][pl.ds(k * L, L)] = lax.shift_left(
                lax.bitwise_and(ids_v, 1), 6
            )
        pltpu.make_async_copy(
            table_hbm.at[rows[b]], gbuf[b], gsems[b]
        ).start()

    def _gather_wait(t, b):
        pltpu.make_async_copy(
            table_hbm.at[rows[b]], gbuf[b], gsems[b]
        ).wait()

    def _out_ref(t):
        s, boff = _chunk_pos(wid, t)
        return out_hbm.at[s, :, pl.ds(boff, C)]

    def _transpose(t, b):
        jvecs = [lax.iota(jnp.int32, L) + k * L for k in range(C // L)]
        cvecs = [cols[b][pl.ds(k * L, L)] for k in range(C // L)]

        def dbody(d, _):
            for k in range(C // L):
                v = plsc.load_gather(gbuf[b], [jvecs[k], cvecs[k] + d])
                tbuf[b][d, pl.ds(k * L, L)] = v
            return _

        lax.fori_loop(0, D, dbody, 0, unroll=2)

    def _write_start(t, b):
        pltpu.make_async_copy(tbuf[b], _out_ref(t), wsems[b]).start()

    def _write_wait(t, b):
        pltpu.make_async_copy(tbuf[b], _out_ref(t), wsems[b]).wait()

    for t in range(A):
        _idx_start(t, t)
        _idx_wait(t, t)
        _gather_start(t, t)

    for t in range(S):
        if t + A < NCH:
            bb = (t + A) % S
            if t >= A:
                _write_wait(t - A, bb)
            _idx_start(t + A, bb)
            _idx_wait(t + A, bb)
            _gather_start(t + A, bb)
        _gather_wait(t, t % S)
        _transpose(t, t % S)
        _write_start(t, t % S)

    def trip(i, _):
        t0 = i * S
        for b in range(S):
            t = t0 + b
            bb = (b + A) % S
            _write_wait(t - A, bb)
            _idx_start(t + A, bb)
            _idx_wait(t + A, bb)
            _gather_start(t + A, bb)
            _gather_wait(t, b)
            _transpose(t, b)
            _write_start(t, b)
        return _

    lax.fori_loop(1, NCH // S - 1, trip, 0)

    t0 = NCH - S
    for b in range(S):
        t = t0 + b
        if t + A < NCH:
            bb = (b + A) % S
            _write_wait(t - A, bb)
            _idx_start(t + A, bb)
            _idx_wait(t + A, bb)
            _gather_start(t + A, bb)
        _gather_wait(t, b)
        _transpose(t, b)
        _write_start(t, b)

    for b in range(S):
        _write_wait(t0 + b, b)


def kernel(token_ids, embedding_matrix):
    ids_flat = token_ids.T.reshape(-1)
    table2 = embedding_matrix.reshape(NUM_EMBEDDINGS // 2, 2 * D)
    out_t = _embed_sc(ids_flat, table2)
    return jnp.transpose(out_t, (2, 0, 1))


# speedup vs baseline: 1.2303x; 1.2303x over previous
import functools

import jax
import jax.numpy as jnp
from jax import lax
from jax.experimental import pallas as pl
from jax.experimental.pallas import tpu as pltpu
from jax.experimental.pallas import tpu_sc as plsc

NUM_EMBEDDINGS = 1000000
D = 64
B_TOTAL = 4096 * 200

NC, NS = 2, 16
NW = NC * NS
L = 16

_PARAMS = pltpu.CompilerParams(
    use_tc_tiling_on_sc=True, needs_layout_passes=False
)
_MESH = plsc.VectorSubcoreMesh(core_axis_name="c", subcore_axis_name="s")


BR = 160
NBLK = NUM_EMBEDDINGS // BR
BPW = -(-NBLK // NW)
RS = 4
RA = 2

assert BPW % RS == 0 and RS == 2 * RA and NBLK * BR == NUM_EMBEDDINGS


@functools.partial(
    pl.kernel,
    out_type=jax.ShapeDtypeStruct((NUM_EMBEDDINGS // 2, 2 * D), jnp.float32),
    mesh=_MESH,
    compiler_params=_PARAMS,
    scratch_types=[
        [pltpu.VMEM((BR, D), jnp.float32) for _ in range(RS)],
        [pltpu.VMEM((BR // 2, 2 * D), jnp.float32) for _ in range(RS)],
        [pltpu.SemaphoreType.DMA for _ in range(RS)],
        [pltpu.SemaphoreType.DMA for _ in range(RS)],
    ],
)
def _repack_sc(tab_hbm, out_hbm, ibufs, obufs, isems, osems):
    wid = lax.axis_index("s") * NC + lax.axis_index("c")
    b0 = wid * BPW

    def _blk(i):
        return jnp.where(b0 + i < NBLK, b0 + i, 0)

    def _in_start(i, s):
        r = pl.multiple_of(_blk(i) * BR, BR)
        pltpu.make_async_copy(tab_hbm.at[pl.ds(r, BR)], ibufs[s], isems[s]).start()

    def _in_wait(i, s):
        r = pl.multiple_of(_blk(i) * BR, BR)
        pltpu.make_async_copy(tab_hbm.at[pl.ds(r, BR)], ibufs[s], isems[s]).wait()

    def _out_start(i, s):
        r = pl.multiple_of(_blk(i) * (BR // 2), BR // 2)
        pltpu.make_async_copy(obufs[s], out_hbm.at[pl.ds(r, BR // 2)], osems[s]).start()

    def _out_wait(i, s):
        r = pl.multiple_of(_blk(i) * (BR // 2), BR // 2)
        pltpu.make_async_copy(obufs[s], out_hbm.at[pl.ds(r, BR // 2)], osems[s]).wait()

    def _pack(s):
        def body(j, _):
            for v in range(D // L):
                obufs[s][j, pl.ds(v * L, L)] = ibufs[s][2 * j, pl.ds(v * L, L)]
                obufs[s][j, pl.ds(D + v * L, L)] = ibufs[s][2 * j + 1, pl.ds(v * L, L)]
            return _
        lax.fori_loop(0, BR // 2, body, 0, unroll=2)

    for i in range(RA):
        _in_start(i, i)
    for i in range(RS):
        if i + RA < BPW:
            bb = (i + RA) % RS
            if i >= RA:
                _out_wait(i - RA, bb)
            _in_start(i + RA, bb)
        _in_wait(i, i % RS)
        _pack(i % RS)
        _out_start(i, i % RS)

    def trip(t, _):
        i0 = t * RS
        for b in range(RS):
            i = i0 + b
            bb = (b + RA) % RS
            _out_wait(i - RA, bb)
            _in_start(i + RA, bb)
            _in_wait(i, b)
            _pack(b)
            _out_start(i, b)
        return _

    lax.fori_loop(1, BPW // RS - 1, trip, 0)

    i0 = BPW - RS
    for b in range(RS):
        i = i0 + b
        if i + RA < BPW:
            bb = (b + RA) % RS
            _out_wait(i - RA, bb)
            _in_start(i + RA, bb)
        _in_wait(i, b)
        _pack(b)
        _out_start(i, b)
    for b in range(RS):
        _out_wait(i0 + b, b)


N_PER_W = B_TOTAL // NW
C = 128
NCH = N_PER_W // C
S = 4
A = 2

assert NCH % S == 0 and S == 2 * A


@functools.partial(
    pl.kernel,
    out_type=jax.ShapeDtypeStruct((B_TOTAL, D), jnp.float32),
    mesh=_MESH,
    compiler_params=_PARAMS,
    scratch_types=[
        [pltpu.VMEM((C,), jnp.int32) for _ in range(2 * S)],
        [pltpu.VMEM((C,), jnp.int32) for _ in range(S)],
        pltpu.VMEM((C,), jnp.int32),
        [pltpu.SemaphoreType.DMA for _ in range(2 * S)],
        [pltpu.VMEM((C, 2 * D), jnp.float32) for _ in range(S)],
        [pltpu.VMEM((C, D), jnp.float32) for _ in range(2)],
        [pltpu.SemaphoreType.DMA for _ in range(S)],
        [pltpu.SemaphoreType.DMA for _ in range(2)],
    ],
)
def _gather_sc(idx_hbm, table_hbm, out_hbm, ixbufs, rowbufs, cb_v, isems,
               gbufs, obufs, gsems, wsems):
    wid = lax.axis_index("s") * NC + lax.axis_index("c")
    base = pl.multiple_of(wid * N_PER_W, N_PER_W)
    IXS = 2 * S

    def _idx_start(g, xs):
        r = pl.multiple_of(base + g * C, C)
        pltpu.make_async_copy(idx_hbm.at[pl.ds(r, C)], ixbufs[xs], isems[xs]).start()

    def _idx_wait(g, xs):
        r = pl.multiple_of(base + g * C, C)
        pltpu.make_async_copy(idx_hbm.at[pl.ds(r, C)], ixbufs[xs], isems[xs]).wait()

    def _gather_start(g, s, xs):
        _idx_wait(g, xs)
        ix = ixbufs[xs]
        def rbody(k, _):
            v = ix[pl.ds(k * L, L)]
            rowbufs[s][pl.ds(k * L, L)] = lax.shift_right_logical(v, 1)
            return _
        lax.fori_loop(0, C // L, rbody, 0, unroll=4)
        pltpu.make_async_copy(
            table_hbm.at[rowbufs[s]], gbufs[s], gsems[s]
        ).start()

    def _gather_wait(g, s):
        pltpu.make_async_copy(
            table_hbm.at[rowbufs[s]], gbufs[s], gsems[s]
        ).wait()

    def _write_start(g, s):
        r = pl.multiple_of(base + g * C, C)
        pltpu.make_async_copy(obufs[s], out_hbm.at[pl.ds(r, C)], wsems[s]).start()

    def _write_wait(g, s):
        r = pl.multiple_of(base + g * C, C)
        pltpu.make_async_copy(obufs[s], out_hbm.at[pl.ds(r, C)], wsems[s]).wait()

    iota = lax.iota(jnp.int32, L)

    def _extract(g, s, so, xs):
        ix = ixbufs[xs]
        def cbody(k, _):
            v = ix[pl.ds(k * L, L)]
            cb_v[pl.ds(k * L, L)] = lax.shift_left(lax.bitwise_and(v, 1), 6)
            return _
        lax.fori_loop(0, C // L, cbody, 0, unroll=4)

        def ebody(j, _):
            cb = plsc.load_gather(cb_v, [jnp.broadcast_to(j, (L,))])
            for v in range(D // L):
                vals = plsc.load_gather(
                    gbufs[s], [jnp.broadcast_to(j, (L,)), cb + (iota + v * L)]
                )
                obufs[so][j, pl.ds(v * L, L)] = vals
            return _
        lax.fori_loop(0, C, ebody, 0, unroll=2)

    P = 2 * S

    for g in range(2 * A):
        _idx_start(g, g % P)
    for g in range(A):
        _gather_start(g, g % S, g % P)
    for g in range(P):
        if g + 2 * A < NCH:
            _idx_start(g + 2 * A, (g + 2 * A) % P)
        if g + A < NCH:
            _gather_start(g + A, (g + A) % S, (g + A) % P)
        _gather_wait(g, g % S)
        if g >= 2:
            _write_wait(g - 2, g % 2)
        _extract(g, g % S, g % 2, g % P)
        _write_start(g, g % 2)

    def trip(i, _):
        g0 = i * P
        for b in range(P):
            g = g0 + b
            _idx_start(g + 2 * A, (b + 2 * A) % P)
            _gather_start(g + A, (b + A) % S, (b + A) % P)
            _gather_wait(g, b % S)
            _write_wait(g - 2, b % 2)
            _extract(g, b % S, b % 2, b)
            _write_start(g, b % 2)
        return _

    lax.fori_loop(1, NCH // P - 1, trip, 0)

    g0 = NCH - P
    for b in range(P):
        g = g0 + b
        if g + 2 * A < NCH:
            _idx_start(g + 2 * A, (b + 2 * A) % P)
        if g + A < NCH:
            _gather_start(g + A, (b + A) % S, (b + A) % P)
        _gather_wait(g, b % S)
        _write_wait(g - 2, b % 2)
        _extract(g, b % S, b % 2, b)
        _write_start(g, b % 2)
    for b in range(2):
        _write_wait(NCH - 2 + b, (NCH - 2 + b) % 2)


def kernel(token_ids, embedding_matrix):
    idx = token_ids.reshape(-1)
    table2 = _repack_sc(embedding_matrix)
    out = _gather_sc(idx, table2)
    return out.reshape(token_ids.shape[0], token_ids.shape[1], D)


# speedup vs baseline: 1.5575x; 1.2660x over previous
import functools

import jax
import jax.numpy as jnp
from jax import lax
from jax.experimental import pallas as pl
from jax.experimental.pallas import tpu as pltpu
from jax.experimental.pallas import tpu_sc as plsc

NUM_EMBEDDINGS = 1000000
D = 64
B_TOTAL = 4096 * 200

NC, NS = 2, 16
NW = NC * NS
N_PER_W = B_TOTAL // NW
C = 128
NCH = N_PER_W // C
S = 8
A = 4

assert NCH % S == 0 and A < S


def _gather_start(table_hbm, idx_v, rows, gsems, g, slot):
    pltpu.make_async_copy(
        table_hbm.at[idx_v.at[pl.ds(g * C, C)]], rows[slot], gsems[slot]
    ).start()


def _gather_wait(table_hbm, idx_v, rows, gsems, g, slot):
    pltpu.make_async_copy(
        table_hbm.at[idx_v.at[pl.ds(g * C, C)]], rows[slot], gsems[slot]
    ).wait()


def _write_start(out_hbm, rows, wsems, base, g, slot):
    pltpu.make_async_copy(
        rows[slot], out_hbm.at[pl.ds(base + g * C, C)], wsems[slot]
    ).start()


def _write_wait(out_hbm, rows, wsems, base, g, slot):
    pltpu.make_async_copy(
        rows[slot], out_hbm.at[pl.ds(base + g * C, C)], wsems[slot]
    ).wait()


@functools.partial(
    pl.kernel,
    out_type=jax.ShapeDtypeStruct((B_TOTAL, D), jnp.float32),
    mesh=plsc.VectorSubcoreMesh(core_axis_name="c", subcore_axis_name="s"),
    compiler_params=pltpu.CompilerParams(use_tc_tiling_on_sc=False),
    scratch_types=[
        pltpu.VMEM((N_PER_W,), jnp.int32),
        [pltpu.VMEM((C, D), jnp.float32) for _ in range(S)],
        [pltpu.SemaphoreType.DMA for _ in range(S)],
        [pltpu.SemaphoreType.DMA for _ in range(S)],
    ],
)
def _embed_sc(idx_hbm, table_hbm, out_hbm, idx_v, rows, gsems, wsems):
    wid = lax.axis_index("s") * NC + lax.axis_index("c")
    base = wid * N_PER_W

    pltpu.sync_copy(idx_hbm.at[pl.ds(base, N_PER_W)], idx_v)

    for g in range(A):
        _gather_start(table_hbm, idx_v, rows, gsems, g, g)

    for g in range(S):
        if g + A < NCH:
            bb = (g + A) % S
            if g >= A:
                _write_wait(out_hbm, rows, wsems, base, g - A, bb)
            _gather_start(table_hbm, idx_v, rows, gsems, g + A, bb)
        _gather_wait(table_hbm, idx_v, rows, gsems, g, g % S)
        _write_start(out_hbm, rows, wsems, base, g, g % S)

    def trip(i, _):
        g0 = i * S
        for b in range(S):
            g = g0 + b
            bb = (b + A) % S
            _write_wait(out_hbm, rows, wsems, base, g - A, bb)
            _gather_start(table_hbm, idx_v, rows, gsems, g + A, bb)
            _gather_wait(table_hbm, idx_v, rows, gsems, g, b)
            _write_start(out_hbm, rows, wsems, base, g, b)
        return _

    lax.fori_loop(1, NCH // S - 1, trip, 0)

    g0 = NCH - S
    for b in range(S):
        g = g0 + b
        if g + A < NCH:
            bb = (b + A) % S
            _write_wait(out_hbm, rows, wsems, base, g - A, bb)
            _gather_start(table_hbm, idx_v, rows, gsems, g + A, bb)
        _gather_wait(table_hbm, idx_v, rows, gsems, g, b)
        _write_start(out_hbm, rows, wsems, base, g, b)

    for b in range(S):
        _write_wait(out_hbm, rows, wsems, base, g0 + b, b)


def kernel(token_ids, embedding_matrix):
    idx = token_ids.reshape(-1)
    out = _embed_sc(idx, embedding_matrix)
    return out.reshape(token_ids.shape[0], token_ids.shape[1], D)
